# baseline (device time: 20710 ns/iter reference)
import jax
import jax.numpy as jnp
from jax import lax
from jax.experimental import pallas as pl
from jax.experimental.pallas import tpu as pltpu

B = 8
H = 8
D = 64
PAGE = 16
NB = 64
NP_LOCAL = 64
KLEN = NP_LOCAL * PAGE
SCALE = D ** -0.5
NEG = -1e30


def kernel(Q, K, V, bt, lens):
    q3 = Q.reshape(B, H, D)
    k3 = K.reshape(KLEN, H, D)
    v3 = V.reshape(KLEN, H, D)
    btT = bt.T.astype(jnp.int32)
    lens2 = lens.reshape(1, B).astype(jnp.int32)

    def body(q_ref, k_ref, v_ref, btT_ref, lens_ref, out_ref,
             c_ref, send_o, recv_o, send_ml, recv_ml,
             sem_so, sem_ro, sem_sml, sem_rml):
        my_x = lax.axis_index("x")
        my_y = lax.axis_index("y")
        peer = (1 - my_x, my_y)

        base = my_x * NP_LOCAL
        pages = base + lax.broadcasted_iota(jnp.int32, (1, NP_LOCAL), 1)
        slot = lax.broadcasted_iota(jnp.int32, (NB, 1), 0)
        for i in range(B):
            bt_col = btT_ref[:, i:i + 1]
            valid = slot < lens_ref[0, i]
            eq = (bt_col == pages) & valid
            c_ref[i:i + 1, :] = jnp.sum(
                eq.astype(jnp.float32), axis=0, keepdims=True)

        cmat = c_ref[:, :]
        logc = jnp.where(cmat > 0.0,
                         jnp.log(jnp.maximum(cmat, 1.0)), NEG)
        r_row = lax.broadcasted_iota(jnp.int32, (NP_LOCAL, KLEN), 0)
        r_col = lax.broadcasted_iota(jnp.int32, (NP_LOCAL, KLEN), 1)
        expander = (r_col // PAGE == r_row).astype(jnp.float32)
        bias = lax.dot_general(
            logc, expander, (((1,), (0,)), ((), ())),
            preferred_element_type=jnp.float32)

        for h in range(H):
            q_h = q_ref[:, h, :].astype(jnp.bfloat16)
            k_h = k_ref[:, h, :].astype(jnp.bfloat16)
            v_h = v_ref[:, h, :].astype(jnp.bfloat16)
            s = lax.dot_general(
                q_h, k_h, (((1,), (1,)), ((), ())),
                preferred_element_type=jnp.float32)
            s = s * SCALE + bias
            m_h = jnp.max(s, axis=1, keepdims=True)
            p = jnp.exp(s - m_h)
            l_h = jnp.sum(p, axis=1, keepdims=True)
            o_h = lax.dot_general(
                p.astype(jnp.bfloat16), v_h, (((1,), (0,)), ((), ())),
                preferred_element_type=jnp.float32)
            send_o[:, h, :] = o_h
            send_ml[0, :, h:h + 1] = m_h
            send_ml[1, :, h:h + 1] = l_h

        bsem = pltpu.get_barrier_semaphore()
        pl.semaphore_signal(bsem, inc=1, device_id=peer,
                            device_id_type=pl.DeviceIdType.MESH)
        pl.semaphore_wait(bsem, 1)

        rdma_o = pltpu.make_async_remote_copy(
            src_ref=send_o, dst_ref=recv_o,
            send_sem=sem_so, recv_sem=sem_ro,
            device_id=peer, device_id_type=pl.DeviceIdType.MESH)
        rdma_ml = pltpu.make_async_remote_copy(
            src_ref=send_ml, dst_ref=recv_ml,
            send_sem=sem_sml, recv_sem=sem_rml,
            device_id=peer, device_id_type=pl.DeviceIdType.MESH)
        rdma_o.start()
        rdma_ml.start()
        rdma_o.wait()
        rdma_ml.wait()

        for h in range(H):
            m_s = send_ml[0, :, h:h + 1]
            l_s = send_ml[1, :, h:h + 1]
            m_p = recv_ml[0, :, h:h + 1]
            l_p = recv_ml[1, :, h:h + 1]
            m = jnp.maximum(m_s, m_p)
            a_s = jnp.exp(m_s - m)
            a_p = jnp.exp(m_p - m)
            l = l_s * a_s + l_p * a_p
            o = (send_o[:, h, :] * a_s + recv_o[:, h, :] * a_p) / l
            out_ref[:, h, :] = o

    out = pl.pallas_call(
        body,
        out_shape=jax.ShapeDtypeStruct((B, H, D), jnp.float32),
        in_specs=[
            pl.BlockSpec(memory_space=pltpu.VMEM),
            pl.BlockSpec(memory_space=pltpu.VMEM),
            pl.BlockSpec(memory_space=pltpu.VMEM),
            pl.BlockSpec(memory_space=pltpu.VMEM),
            pl.BlockSpec(memory_space=pltpu.SMEM),
        ],
        out_specs=pl.BlockSpec(memory_space=pltpu.VMEM),
        scratch_shapes=[
            pltpu.VMEM((B, NP_LOCAL), jnp.float32),
            pltpu.VMEM((B, H, D), jnp.float32),
            pltpu.VMEM((B, H, D), jnp.float32),
            pltpu.VMEM((2, B, H), jnp.float32),
            pltpu.VMEM((2, B, H), jnp.float32),
            pltpu.SemaphoreType.DMA,
            pltpu.SemaphoreType.DMA,
            pltpu.SemaphoreType.DMA,
            pltpu.SemaphoreType.DMA,
        ],
        compiler_params=pltpu.CompilerParams(collective_id=0),
    )(q3, k3, v3, btT, lens2)
    return out.reshape(B, 1, H, D)


# device time: 20649 ns/iter; 1.0030x vs baseline; 1.0030x over previous
import jax
import jax.numpy as jnp
from jax import lax
from jax.experimental import pallas as pl
from jax.experimental.pallas import tpu as pltpu

B = 8
H = 8
D = 64
PAGE = 16
NB = 64
NP_LOCAL = 64
KLEN = NP_LOCAL * PAGE
SCALE = D ** -0.5
NEG = -1e30


def kernel(Q, K, V, bt, lens):
    lens2 = lens.reshape(1, B).astype(jnp.int32)

    def body(q_ref, k_ref, v_ref, bt_ref, lens_ref, out_ref,
             c_ref, send_o, recv_o, send_ml, recv_ml,
             sem_so, sem_ro, sem_sml, sem_rml):
        my_x = lax.axis_index("x")
        my_y = lax.axis_index("y")
        peer = (1 - my_x, my_y)

        base = my_x * NP_LOCAL
        pages_col = base + lax.broadcasted_iota(jnp.int32, (NP_LOCAL, 1), 0)
        slot_row = lax.broadcasted_iota(jnp.int32, (1, NB), 1)
        for i in range(B):
            bt_row = bt_ref[i:i + 1, :]
            valid = slot_row < lens_ref[0, i]
            eq = (pages_col == bt_row) & valid
            c_ref[:, i:i + 1] = jnp.sum(
                eq.astype(jnp.float32), axis=1, keepdims=True)

        cmat = c_ref[:, :]
        logc = jnp.where(cmat > 0.0,
                         jnp.log(jnp.maximum(cmat, 1.0)), NEG)
        r_row = lax.broadcasted_iota(jnp.int32, (NP_LOCAL, KLEN), 0)
        r_col = lax.broadcasted_iota(jnp.int32, (NP_LOCAL, KLEN), 1)
        expander = (r_col // PAGE == r_row).astype(jnp.float32)
        bias = lax.dot_general(
            logc, expander, (((0,), (0,)), ((), ())),
            preferred_element_type=jnp.float32)

        for h in range(H):
            q_h = q_ref[:, 0, h, :].astype(jnp.bfloat16)
            k_h = k_ref[:, :, h, :].astype(jnp.bfloat16).reshape(KLEN, D)
            v_h = v_ref[:, :, h, :].astype(jnp.bfloat16).reshape(KLEN, D)
            s = lax.dot_general(
                q_h, k_h, (((1,), (1,)), ((), ())),
                preferred_element_type=jnp.float32)
            s = s * SCALE + bias
            m_h = jnp.max(s, axis=1, keepdims=True)
            p = jnp.exp(s - m_h)
            l_h = jnp.sum(p, axis=1, keepdims=True)
            o_h = lax.dot_general(
                p.astype(jnp.bfloat16), v_h, (((1,), (0,)), ((), ())),
                preferred_element_type=jnp.float32)
            send_o[:, h, :] = o_h
            send_ml[0, :, h:h + 1] = m_h
            send_ml[1, :, h:h + 1] = l_h

        bsem = pltpu.get_barrier_semaphore()
        pl.semaphore_signal(bsem, inc=1, device_id=peer,
                            device_id_type=pl.DeviceIdType.MESH)
        pl.semaphore_wait(bsem, 1)

        rdma_o = pltpu.make_async_remote_copy(
            src_ref=send_o, dst_ref=recv_o,
            send_sem=sem_so, recv_sem=sem_ro,
            device_id=peer, device_id_type=pl.DeviceIdType.MESH)
        rdma_ml = pltpu.make_async_remote_copy(
            src_ref=send_ml, dst_ref=recv_ml,
            send_sem=sem_sml, recv_sem=sem_rml,
            device_id=peer, device_id_type=pl.DeviceIdType.MESH)
        rdma_o.start()
        rdma_ml.start()
        rdma_o.wait()
        rdma_ml.wait()

        for h in range(H):
            m_s = send_ml[0, :, h:h + 1]
            l_s = send_ml[1, :, h:h + 1]
            m_p = recv_ml[0, :, h:h + 1]
            l_p = recv_ml[1, :, h:h + 1]
            m = jnp.maximum(m_s, m_p)
            a_s = jnp.exp(m_s - m)
            a_p = jnp.exp(m_p - m)
            l = l_s * a_s + l_p * a_p
            o = (send_o[:, h, :] * a_s + recv_o[:, h, :] * a_p) / l
            out_ref[:, 0, h, :] = o

    return pl.pallas_call(
        body,
        out_shape=jax.ShapeDtypeStruct((B, 1, H, D), jnp.float32),
        in_specs=[
            pl.BlockSpec(memory_space=pltpu.VMEM),
            pl.BlockSpec(memory_space=pltpu.VMEM),
            pl.BlockSpec(memory_space=pltpu.VMEM),
            pl.BlockSpec(memory_space=pltpu.VMEM),
            pl.BlockSpec(memory_space=pltpu.SMEM),
        ],
        out_specs=pl.BlockSpec(memory_space=pltpu.VMEM),
        scratch_shapes=[
            pltpu.VMEM((NP_LOCAL, B), jnp.float32),
            pltpu.VMEM((B, H, D), jnp.float32),
            pltpu.VMEM((B, H, D), jnp.float32),
            pltpu.VMEM((2, B, H), jnp.float32),
            pltpu.VMEM((2, B, H), jnp.float32),
            pltpu.SemaphoreType.DMA,
            pltpu.SemaphoreType.DMA,
            pltpu.SemaphoreType.DMA,
            pltpu.SemaphoreType.DMA,
        ],
        compiler_params=pltpu.CompilerParams(collective_id=0),
    )(Q, K, V, bt, lens2)


# device time: 10419 ns/iter; 1.9877x vs baseline; 1.9819x over previous
import jax
import jax.numpy as jnp
from jax import lax
from jax.experimental import pallas as pl
from jax.experimental.pallas import tpu as pltpu

B = 8
H = 8
D = 64
PAGE = 16
NB = 64
NP_SHARD = 64
NP_OWN = 32
KOWN = NP_OWN * PAGE
KH = NP_OWN * PAGE * H
R = B * H
SCALE = D ** -0.5
NEG = -1e30


def kernel(Q, K, V, bt, lens):
    my_y_out = lax.axis_index("y")
    k2 = lax.dynamic_slice_in_dim(
        K, my_y_out * NP_OWN, NP_OWN, axis=0).reshape(KH, D)
    v2 = lax.dynamic_slice_in_dim(
        V, my_y_out * NP_OWN, NP_OWN, axis=0).reshape(KH, D)
    q2 = Q.reshape(R, D)
    lens2 = lens.reshape(1, B).astype(jnp.int32)

    def body(q_ref, k_ref, v_ref, bt_ref, lens_ref, out_ref,
             c_ref, send_buf, recv_buf,
             send_sems, recv_sems):
        my_x = lax.axis_index("x")
        my_y = lax.axis_index("y")
        peers = [(1 - my_x, my_y), (my_x, 1 - my_y), (1 - my_x, 1 - my_y)]

        bsem = pltpu.get_barrier_semaphore()
        for pr in peers:
            pl.semaphore_signal(bsem, inc=1, device_id=pr,
                                device_id_type=pl.DeviceIdType.MESH)

        base = my_x * NP_SHARD + my_y * NP_OWN
        pages_col = base + lax.broadcasted_iota(jnp.int32, (NP_OWN, 1), 0)
        slot_row = lax.broadcasted_iota(jnp.int32, (1, NB), 1)
        for i in range(B):
            bt_row = bt_ref[i:i + 1, :]
            valid = slot_row < lens_ref[0, i]
            eq = (pages_col == bt_row) & valid
            c_ref[:, i:i + 1] = jnp.sum(
                eq.astype(jnp.float32), axis=1, keepdims=True)

        logc = jnp.where(c_ref[:, :] > 0.0,
                         jnp.log(jnp.maximum(c_ref[:, :], 1.0)),
                         NEG)

        def io(shape, dim):
            return lax.broadcasted_iota(jnp.int32, shape, dim)

        rep = (io((R, B), 0) // H == io((R, B), 1))
        erho = (io((NP_OWN, KH), 1) // (PAGE * H)
                == io((NP_OWN, KH), 0))
        onehot = (io((R, H), 0) % H == io((R, H), 1))
        neq = (io((H, KH), 1) % H != io((H, KH), 0))

        pagebias = lax.dot_general(
            rep.astype(jnp.float32), logc, (((1,), (1,)), ((), ())),
            preferred_element_type=jnp.float32)
        lhs = jnp.concatenate(
            [pagebias, onehot.astype(jnp.float32) * NEG], axis=1)
        rhs = jnp.concatenate(
            [erho.astype(jnp.float32), neq.astype(jnp.float32)],
            axis=0)
        bias = lax.dot_general(
            lhs, rhs, (((1,), (0,)), ((), ())),
            preferred_element_type=jnp.float32)

        s = lax.dot_general(
            (q_ref[:, :] * SCALE).astype(jnp.bfloat16),
            k_ref[:, :].astype(jnp.bfloat16),
            (((1,), (1,)), ((), ())),
            preferred_element_type=jnp.float32)
        s = s + bias

        m = jnp.max(s, axis=1, keepdims=True)
        p = jnp.exp(s - m)
        l = jnp.sum(p, axis=1, keepdims=True)

        o = lax.dot_general(
            p.astype(jnp.bfloat16), v_ref[:, :].astype(jnp.bfloat16),
            (((1,), (0,)), ((), ())),
            preferred_element_type=jnp.float32)

        send_buf[:, 0:D] = o
        send_buf[:, D:D + 1] = m
        send_buf[:, D + 1:D + 2] = l

        pl.semaphore_wait(bsem, 3)

        rdmas = []
        for i, pr in enumerate(peers):
            r = pltpu.make_async_remote_copy(
                src_ref=send_buf, dst_ref=recv_buf.at[i],
                send_sem=send_sems.at[i], recv_sem=recv_sems.at[i],
                device_id=pr, device_id_type=pl.DeviceIdType.MESH)
            r.start()
            rdmas.append(r)
        for r in rdmas:
            r.wait()

        ms = [m] + [recv_buf[i, :, D:D + 1] for i in range(3)]
        ls = [l] + [recv_buf[i, :, D + 1:D + 2] for i in range(3)]
        os_ = [o] + [recv_buf[i, :, 0:D] for i in range(3)]
        mm = jnp.maximum(jnp.maximum(ms[0], ms[1]),
                         jnp.maximum(ms[2], ms[3]))
        lsum = jnp.zeros_like(l)
        osum = jnp.zeros_like(o)
        for mi, li, oi in zip(ms, ls, os_):
            ai = jnp.exp(mi - mm)
            lsum = lsum + li * ai
            osum = osum + oi * ai
        out_ref[:, 0, :, :] = (osum / lsum).reshape(B, H, D)

    return pl.pallas_call(
        body,
        out_shape=jax.ShapeDtypeStruct((B, 1, H, D), jnp.float32),
        in_specs=[
            pl.BlockSpec(memory_space=pltpu.VMEM),
            pl.BlockSpec(memory_space=pltpu.VMEM),
            pl.BlockSpec(memory_space=pltpu.VMEM),
            pl.BlockSpec(memory_space=pltpu.VMEM),
            pl.BlockSpec(memory_space=pltpu.SMEM),
        ],
        out_specs=pl.BlockSpec(memory_space=pltpu.VMEM),
        scratch_shapes=[
            pltpu.VMEM((NP_OWN, B), jnp.float32),
            pltpu.VMEM((R, D + 2), jnp.float32),
            pltpu.VMEM((3, R, D + 2), jnp.float32),
            pltpu.SemaphoreType.DMA((3,)),
            pltpu.SemaphoreType.DMA((3,)),
        ],
        compiler_params=pltpu.CompilerParams(collective_id=0),
    )(q2, k2, v2, bt, lens2)
